# baseline (speedup 1.0000x reference)
import jax
import jax.numpy as jnp
from jax.experimental import pallas as pl
from jax.experimental.pallas import tpu as pltpu

_ROWS = 2048
_K = 3


def _router_kernel(x_ref, w_ref, b_ref, mask_ref, prob_ref, imp_ref):
    xf = x_ref[...]
    wf = w_ref[...]
    xh = xf.astype(jnp.bfloat16)
    xl = (xf - xh.astype(jnp.float32)).astype(jnp.bfloat16)
    wh = wf.astype(jnp.bfloat16)
    wl = (wf - wh.astype(jnp.float32)).astype(jnp.bfloat16)

    def _mm(a, c):
        return jnp.dot(a, c, preferred_element_type=jnp.float32)

    logits = ((_mm(xh, wh) + _mm(xl, wh)) + _mm(xh, wl)) + b_ref[...]

    m = jnp.max(logits, axis=-1, keepdims=True)
    e = jnp.exp(logits - m)
    prob = e / jnp.sum(e, axis=-1, keepdims=True)
    prob_ref[...] = prob

    @pl.when(pl.program_id(0) == 0)
    def _init():
        imp_ref[...] = jnp.zeros_like(imp_ref)

    imp_ref[...] += jnp.sum(prob, axis=0, keepdims=True)

    n_e = logits.shape[-1]
    cols = jax.lax.broadcasted_iota(
        jnp.int32, logits.shape, 1).astype(jnp.float32)
    big = jnp.float32(n_e)
    work = logits
    hit_any = None
    for _ in range(_K):
        mx = jnp.max(work, axis=-1, keepdims=True)
        cand = jnp.where(work == mx, cols, big)
        sel = jnp.min(cand, axis=-1, keepdims=True)
        hit = cols == sel
        hit_any = hit if hit_any is None else (hit_any | hit)
        work = jnp.where(hit, -jnp.inf, work)
    mask_ref[...] = hit_any.astype(jnp.float32)


def kernel(x, W, b):
    x = x.reshape(x.shape[0], -1)
    n, d = x.shape
    n_e = W.shape[1]
    grid = n // _ROWS
    mask, prob, imp = pl.pallas_call(
        _router_kernel,
        grid=(grid,),
        in_specs=[
            pl.BlockSpec((_ROWS, d), lambda i: (i, 0)),
            pl.BlockSpec((d, n_e), lambda i: (0, 0)),
            pl.BlockSpec((1, n_e), lambda i: (0, 0)),
        ],
        out_specs=[
            pl.BlockSpec((_ROWS, n_e), lambda i: (i, 0)),
            pl.BlockSpec((_ROWS, n_e), lambda i: (i, 0)),
            pl.BlockSpec((1, n_e), lambda i: (0, 0)),
        ],
        out_shape=[
            jax.ShapeDtypeStruct((n, n_e), jnp.float32),
            jax.ShapeDtypeStruct((n, n_e), jnp.float32),
            jax.ShapeDtypeStruct((1, n_e), jnp.float32),
        ],
        compiler_params=pltpu.CompilerParams(
            dimension_semantics=("arbitrary",)),
    )(x, W, b.reshape(1, -1))
    imp = imp.reshape(-1)
    return mask, prob, imp, imp


# speedup vs baseline: 1.2535x; 1.2535x over previous
import jax
import jax.numpy as jnp
from jax.experimental import pallas as pl
from jax.experimental.pallas import tpu as pltpu

_ROWS = 2048
_K = 3


def _router_kernel(x_ref, w_ref, b_ref, mask_ref, prob_ref, imp_ref,
                   lg_ref):
    i = pl.program_id(0)
    nblocks = pl.num_programs(0) - 1

    @pl.when(i < nblocks)
    def _matmul():
        cur = jax.lax.rem(i, 2)
        lg_ref[pl.ds(cur, 1)] = jnp.dot(
            x_ref[...], w_ref[...],
            preferred_element_type=jnp.float32)[None]

    @pl.when(i > 0)
    def _post():
        prev = jax.lax.rem(i + 1, 2)
        logits = lg_ref[pl.ds(prev, 1)][0] + b_ref[...]

        m = jnp.max(logits, axis=-1, keepdims=True)
        e = jnp.exp(logits - m)
        prob = e / jnp.sum(e, axis=-1, keepdims=True)
        prob_ref[...] = prob

        psum = jnp.sum(prob, axis=0, keepdims=True)
        imp_prev = jnp.where(i == 1, jnp.zeros_like(imp_ref[...]),
                             imp_ref[...])
        imp_ref[...] = imp_prev + psum

        n_e = logits.shape[-1]
        cols = jax.lax.broadcasted_iota(
            jnp.int32, logits.shape, 1).astype(jnp.float32)
        big = jnp.float32(n_e)
        work = logits
        hit_any = None
        for _ in range(_K):
            mx = jnp.max(work, axis=-1, keepdims=True)
            cand = jnp.where(work == mx, cols, big)
            sel = jnp.min(cand, axis=-1, keepdims=True)
            hit = cols == sel
            hit_any = hit if hit_any is None else (hit_any | hit)
            work = jnp.where(hit, -jnp.inf, work)
        mask_ref[...] = hit_any.astype(jnp.float32)


def kernel(x, W, b):
    x = x.reshape(x.shape[0], -1)
    n, d = x.shape
    n_e = W.shape[1]
    grid = n // _ROWS
    last = grid - 1
    mask, prob, imp = pl.pallas_call(
        _router_kernel,
        grid=(grid + 1,),
        in_specs=[
            pl.BlockSpec((_ROWS, d), lambda i: (jnp.minimum(i, last), 0)),
            pl.BlockSpec((d, n_e), lambda i: (0, 0)),
            pl.BlockSpec((1, n_e), lambda i: (0, 0)),
        ],
        out_specs=[
            pl.BlockSpec((_ROWS, n_e),
                         lambda i: (jnp.maximum(i - 1, 0), 0)),
            pl.BlockSpec((_ROWS, n_e),
                         lambda i: (jnp.maximum(i - 1, 0), 0)),
            pl.BlockSpec((1, n_e), lambda i: (0, 0)),
        ],
        out_shape=[
            jax.ShapeDtypeStruct((n, n_e), jnp.float32),
            jax.ShapeDtypeStruct((n, n_e), jnp.float32),
            jax.ShapeDtypeStruct((1, n_e), jnp.float32),
        ],
        scratch_shapes=[pltpu.VMEM((2, _ROWS, n_e), jnp.float32)],
        compiler_params=pltpu.CompilerParams(
            dimension_semantics=("arbitrary",)),
    )(x, W, b.reshape(1, -1))
    imp = imp.reshape(-1)
    return mask, prob, imp, imp


# speedup vs baseline: 1.3450x; 1.0730x over previous
import jax
import jax.numpy as jnp
from jax.experimental import pallas as pl
from jax.experimental.pallas import tpu as pltpu

_ROWS = 1024
_K = 3


def _router_kernel(x_ref, w_ref, b_ref, mask_ref, prob_ref, imp_ref):
    logits = jnp.dot(x_ref[...], w_ref[...],
                     preferred_element_type=jnp.float32) + b_ref[...]

    m = jnp.max(logits, axis=-1, keepdims=True)
    e = jnp.exp(logits - m)
    prob = e / jnp.sum(e, axis=-1, keepdims=True)
    prob_ref[...] = prob

    @pl.when(pl.program_id(0) == 0)
    def _init():
        imp_ref[...] = jnp.zeros_like(imp_ref)

    imp_ref[...] += jnp.sum(prob, axis=0, keepdims=True)

    n_e = logits.shape[-1]
    cols = jax.lax.broadcasted_iota(
        jnp.int32, logits.shape, 1).astype(jnp.float32)
    big = jnp.float32(n_e)
    work = logits
    hit_any = None
    for _ in range(_K):
        mx = jnp.max(work, axis=-1, keepdims=True)
        cand = jnp.where(work == mx, cols, big)
        sel = jnp.min(cand, axis=-1, keepdims=True)
        hit = cols == sel
        hit_any = hit if hit_any is None else (hit_any | hit)
        work = jnp.where(hit, -jnp.inf, work)
    mask_ref[...] = hit_any.astype(jnp.float32)


def kernel(x, W, b):
    x = x.reshape(x.shape[0], -1)
    n, d = x.shape
    n_e = W.shape[1]
    grid = n // _ROWS
    mask, prob, imp = pl.pallas_call(
        _router_kernel,
        grid=(grid,),
        in_specs=[
            pl.BlockSpec((_ROWS, d), lambda i: (i, 0)),
            pl.BlockSpec((d, n_e), lambda i: (0, 0)),
            pl.BlockSpec((1, n_e), lambda i: (0, 0)),
        ],
        out_specs=[
            pl.BlockSpec((_ROWS, n_e), lambda i: (i, 0)),
            pl.BlockSpec((_ROWS, n_e), lambda i: (i, 0)),
            pl.BlockSpec((1, n_e), lambda i: (0, 0)),
        ],
        out_shape=[
            jax.ShapeDtypeStruct((n, n_e), jnp.float32),
            jax.ShapeDtypeStruct((n, n_e), jnp.float32),
            jax.ShapeDtypeStruct((1, n_e), jnp.float32),
        ],
        compiler_params=pltpu.CompilerParams(
            dimension_semantics=("arbitrary",)),
    )(x, W, b.reshape(1, -1))
    imp = imp.reshape(-1)
    return mask, prob, imp, imp


# speedup vs baseline: 1.3552x; 1.0076x over previous
import jax
import jax.numpy as jnp
from jax.experimental import pallas as pl
from jax.experimental.pallas import tpu as pltpu

_ROWS = 2048
_K = 3


def _router_kernel(x_ref, w_ref, b_ref, mask_ref, prob_ref, imp_ref):
    logits = jnp.dot(x_ref[...], w_ref[...],
                     preferred_element_type=jnp.float32) + b_ref[...]

    m = jnp.max(logits, axis=-1, keepdims=True)
    e = jnp.exp(logits - m)
    prob = e / jnp.sum(e, axis=-1, keepdims=True)
    prob_ref[...] = prob

    @pl.when(pl.program_id(0) == 0)
    def _init():
        imp_ref[...] = jnp.zeros_like(imp_ref)

    imp_ref[...] += jnp.sum(prob, axis=0, keepdims=True)

    n_e = logits.shape[-1]
    cols = jax.lax.broadcasted_iota(
        jnp.int32, logits.shape, 1).astype(jnp.float32)
    big = jnp.float32(n_e)
    work = logits
    hit_any = None
    for _ in range(_K):
        mx = jnp.max(work, axis=-1, keepdims=True)
        cand = jnp.where(work == mx, cols, big)
        sel = jnp.min(cand, axis=-1, keepdims=True)
        hit = cols == sel
        hit_any = hit if hit_any is None else (hit_any | hit)
        work = jnp.where(hit, -jnp.inf, work)
    mask_ref[...] = hit_any.astype(jnp.float32)


def kernel(x, W, b):
    x = x.reshape(x.shape[0], -1)
    n, d = x.shape
    n_e = W.shape[1]
    grid = n // _ROWS
    mask, prob, imp = pl.pallas_call(
        _router_kernel,
        grid=(grid,),
        in_specs=[
            pl.BlockSpec((_ROWS, d), lambda i: (i, 0)),
            pl.BlockSpec((d, n_e), lambda i: (0, 0)),
            pl.BlockSpec((1, n_e), lambda i: (0, 0)),
        ],
        out_specs=[
            pl.BlockSpec((_ROWS, n_e), lambda i: (i, 0)),
            pl.BlockSpec((_ROWS, n_e), lambda i: (i, 0)),
            pl.BlockSpec((1, n_e), lambda i: (0, 0)),
        ],
        out_shape=[
            jax.ShapeDtypeStruct((n, n_e), jnp.float32),
            jax.ShapeDtypeStruct((n, n_e), jnp.float32),
            jax.ShapeDtypeStruct((1, n_e), jnp.float32),
        ],
        compiler_params=pltpu.CompilerParams(
            dimension_semantics=("arbitrary",)),
    )(x, W, b.reshape(1, -1))
    imp = imp.reshape(-1)
    return mask, prob, imp, imp
